# baseline (speedup 1.0000x reference)
import functools

import jax
import jax.numpy as jnp
import numpy as np
from jax import lax
from jax.experimental import pallas as pl
from jax.experimental.pallas import tpu as pltpu
from jax.experimental.pallas import tpu_sc as plsc

E = 8
D_MODEL = 1024
D_FF = 2048
T = 1024
NC = 2
NSUB = 16
L = 16
NW = NC * NSUB
TPW = T // NW
BM = 128
NB = (2 * T) // BM + E - 1
NSLOT = NB * BM

_MESH = plsc.VectorSubcoreMesh(core_axis_name="c", subcore_axis_name="s")


def _ones01():
    return jnp.full((L,), 1, jnp.int32), jnp.zeros((L,), jnp.int32)


def _wid():
    return lax.axis_index("s") * NC + lax.axis_index("c")


def _splat(v, i):
    idx = jnp.full((L,), i, jnp.int32)
    return v.at[idx].get(mode="promise_in_bounds")


def _cumsum(v):
    iota = lax.iota(jnp.int32, L)
    zero = jnp.zeros((L,), v.dtype)
    acc = v
    for d in (1, 2, 4, 8):
        sh = acc.at[jnp.maximum(iota - d, 0)].get(mode="promise_in_bounds")
        acc = acc + jnp.where(iota >= d, sh, zero)
    return acc


def _cummax(v):
    iota = lax.iota(jnp.int32, L)
    acc = v
    for d in (1, 2, 4, 8):
        sh = acc.at[jnp.maximum(iota - d, 0)].get(mode="promise_in_bounds")
        acc = jnp.where(iota >= d, jnp.maximum(acc, sh), acc)
    return acc


def _routing_groups(gv_ref):
    out = []
    for g in range(2):
        ge = [gv_ref[pl.ds(e * TPW + g * L, L)] for e in range(E)]
        m1 = ge[0]
        i1 = jnp.zeros((L,), jnp.int32)
        for e in range(1, E):
            upd = ge[e] > m1
            m1 = jnp.where(upd, ge[e], m1)
            i1 = jnp.where(upd, e, i1)
        m2 = jnp.full((L,), -jnp.inf, jnp.float32)
        i2 = jnp.zeros((L,), jnp.int32)
        for e in range(E):
            cand = jnp.where(i1 == e, -jnp.inf, ge[e])
            upd = cand > m2
            m2 = jnp.where(upd, cand, m2)
            i2 = jnp.where(upd, e, i2)
        wa = 1.0 / (1.0 + jnp.exp(m2 - m1))
        out.append((i1, i2, wa, 1.0 - wa))
    return out


@functools.partial(
    pl.kernel,
    out_type=jax.ShapeDtypeStruct((NW * L,), jnp.int32),
    mesh=_MESH,
    scratch_types=[
        pltpu.VMEM((E * TPW,), jnp.float32),
        pltpu.VMEM((L,), jnp.int32),
    ],
)
def _k1a_counts(gt_hbm, counts_hbm, gv, cv):
    w = _wid()
    for e in range(E):
        pltpu.sync_copy(gt_hbm.at[pl.ds(e * T + w * TPW, TPW)],
                        gv.at[pl.ds(e * TPW, TPW)])
    groups = _routing_groups(gv)
    iota = lax.iota(jnp.int32, L)
    counts = jnp.zeros((L,), jnp.int32)
    one, zero = _ones01()
    for (i1, i2, _, _) in groups:
        for v in (i1, i2):
            for a in range(L):
                counts = counts + jnp.where(_splat(v, a) == iota, one, zero)
    cv[...] = counts
    pltpu.sync_copy(cv, counts_hbm.at[pl.ds(w * L, L)])


@functools.partial(
    pl.kernel,
    out_type=[
        jax.ShapeDtypeStruct((NSLOT, D_MODEL), jnp.float32),
        jax.ShapeDtypeStruct((NW,), jnp.int32),
        jax.ShapeDtypeStruct((T,), jnp.int32),
        jax.ShapeDtypeStruct((T,), jnp.int32),
        jax.ShapeDtypeStruct((T,), jnp.float32),
        jax.ShapeDtypeStruct((T,), jnp.float32),
    ],
    mesh=_MESH,
    scratch_types=[
        pltpu.VMEM((E * TPW,), jnp.float32),
        pltpu.VMEM((NW * L,), jnp.int32),
        pltpu.VMEM((TPW, D_MODEL), jnp.float32),
        pltpu.VMEM((TPW,), jnp.int32),
        pltpu.VMEM((TPW,), jnp.int32),
        pltpu.VMEM((TPW,), jnp.float32),
        pltpu.VMEM((TPW,), jnp.float32),
        pltpu.VMEM((NW,), jnp.int32),
        pltpu.SemaphoreType.DMA,
    ],
)
def _k1b_dispatch(gt_hbm, counts_hbm, x_hbm,
                  xs_hbm, be_hbm, posa_hbm, posb_hbm, wa_hbm, wb_hbm,
                  gv, cnts, xrows, pA, pB, wav, wbv, bev, sem):
    w = _wid()
    iota = lax.iota(jnp.int32, L)
    for e in range(E):
        pltpu.sync_copy(gt_hbm.at[pl.ds(e * T + w * TPW, TPW)],
                        gv.at[pl.ds(e * TPW, TPW)])
    pltpu.sync_copy(counts_hbm, cnts)
    groups = _routing_groups(gv)

    def _pref(i, acc):
        return acc + cnts[pl.ds(i * L, L)]

    excl = lax.fori_loop(0, w, _pref, jnp.zeros((L,), jnp.int32))
    c_tot = excl + lax.fori_loop(w, NW, _pref, jnp.zeros((L,), jnp.int32))

    nb = (c_tot + (BM - 1)) >> 7
    startblk = _cumsum(nb) - nb
    off = startblk * BM + excl

    for g in range(2):
        i1, i2, wa, wb = groups[g]
        wav[pl.ds(g * L, L)] = wa
        wbv[pl.ds(g * L, L)] = wb
        posg = []
        for idx in (i1, i2):
            pos = jnp.zeros((L,), jnp.int32)
            one, zero = _ones01()
            for e in range(E):
                m = idx == e
                r = _cumsum(jnp.where(m, one, zero))
                pos = jnp.where(m, _splat(off, e) + r - 1, pos)
                off = jnp.where(iota == e, off + _splat(r, L - 1), off)
            posg.append(pos)
        pA[pl.ds(g * L, L)] = posg[0]
        pB[pl.ds(g * L, L)] = posg[1]

    base = w * TPW
    pltpu.sync_copy(wav, wa_hbm.at[pl.ds(base, TPW)])
    pltpu.sync_copy(wbv, wb_hbm.at[pl.ds(base, TPW)])
    pltpu.sync_copy(pA, posa_hbm.at[pl.ds(base, TPW)])
    pltpu.sync_copy(pB, posb_hbm.at[pl.ds(base, TPW)])

    pltpu.sync_copy(x_hbm.at[pl.ds(base, TPW)], xrows)
    pltpu.async_copy(xrows, xs_hbm.at[pA], sem).wait()
    pltpu.async_copy(xrows, xs_hbm.at[pB], sem).wait()

    used_max = _cummax(jnp.where(nb > 0, iota, jnp.zeros((L,), jnp.int32)))
    endblk = startblk + nb
    for h in range(2):
        bi = lax.iota(jnp.int32, L) + h * L
        accb = _splat(used_max, L - 1)
        for e in range(E):
            inb = (bi >= _splat(startblk, e)) & (bi < _splat(endblk, e))
            accb = jnp.where(inb, e, accb)
        bev[pl.ds(h * L, L)] = accb

    @pl.when(w == 0)
    def _():
        pltpu.sync_copy(bev, be_hbm)


def _k2_body(be_ref, xs_ref, w1g_ref, w1u_ref, w2_ref, ys_ref):
    f = pl.program_id(0)
    i = pl.program_id(1)
    xb = xs_ref[...]
    g = jnp.dot(xb, w1g_ref[0], preferred_element_type=jnp.float32)
    u = jnp.dot(xb, w1u_ref[0], preferred_element_type=jnp.float32)
    a = g * jax.lax.logistic(g) * u
    y = jnp.dot(a, w2_ref[0], preferred_element_type=jnp.float32)
    rows = pl.ds(i * BM, BM)

    @pl.when(f == 0)
    def _():
        ys_ref[rows, :] = y

    @pl.when(f != 0)
    def _():
        ys_ref[rows, :] += y


@functools.partial(
    pl.kernel,
    out_type=jax.ShapeDtypeStruct((T, D_MODEL), jnp.float32),
    mesh=_MESH,
    scratch_types=[
        pltpu.VMEM((TPW,), jnp.int32),
        pltpu.VMEM((TPW,), jnp.int32),
        pltpu.VMEM((TPW,), jnp.float32),
        pltpu.VMEM((TPW,), jnp.float32),
        pltpu.VMEM((TPW, D_MODEL), jnp.float32),
        pltpu.VMEM((TPW, D_MODEL), jnp.float32),
        pltpu.VMEM((TPW, D_MODEL), jnp.float32),
        pltpu.SemaphoreType.DMA,
    ],
)
def _k3_combine(ys_hbm, posa_hbm, posb_hbm, wa_hbm, wb_hbm, out_hbm,
                pA, pB, wav, wbv, rowsA, rowsB, orows, sem):
    w = _wid()
    base = w * TPW
    pltpu.sync_copy(posa_hbm.at[pl.ds(base, TPW)], pA)
    pltpu.sync_copy(posb_hbm.at[pl.ds(base, TPW)], pB)
    pltpu.sync_copy(wa_hbm.at[pl.ds(base, TPW)], wav)
    pltpu.sync_copy(wb_hbm.at[pl.ds(base, TPW)], wbv)
    pltpu.async_copy(ys_hbm.at[pA], rowsA, sem).wait()
    pltpu.async_copy(ys_hbm.at[pB], rowsB, sem).wait()
    for t in range(TPW):
        wac = wav[pl.ds((t // L) * L, L)]
        wbc = wbv[pl.ds((t // L) * L, L)]
        sa = _splat(wac, t % L)
        sb = _splat(wbc, t % L)

        def _chunks(c, _, t=t, sa=sa, sb=sb):
            for k in range(8):
                s = pl.ds(c * (8 * L) + k * L, L)
                orows[t, s] = sa * rowsA[t, s] + sb * rowsB[t, s]
            return 0

        lax.fori_loop(0, D_MODEL // (8 * L), _chunks, 0)
    pltpu.sync_copy(orows, out_hbm.at[pl.ds(base, TPW)])


def kernel(x, gating_output, w1, w2):
    gt = gating_output.T.reshape(-1)
    counts = _k1a_counts(gt)
    xs, be, posa, posb, wa, wb = _k1b_dispatch(gt, counts, x)

    grid_spec = pltpu.PrefetchScalarGridSpec(
        num_scalar_prefetch=1,
        grid=(2, NB),
        in_specs=[
            pl.BlockSpec((BM, D_MODEL), lambda f, i, be_ref: (i, 0)),
            pl.BlockSpec((1, D_MODEL, D_FF // 2),
                         lambda f, i, be_ref: (be_ref[i], 0, f)),
            pl.BlockSpec((1, D_MODEL, D_FF // 2),
                         lambda f, i, be_ref: (be_ref[i], 0, 2 + f)),
            pl.BlockSpec((1, D_FF // 2, D_MODEL),
                         lambda f, i, be_ref: (be_ref[i], f, 0)),
        ],
        out_specs=pl.BlockSpec((NSLOT, D_MODEL), lambda f, i, be_ref: (0, 0)),
    )
    ys = pl.pallas_call(
        _k2_body,
        grid_spec=grid_spec,
        out_shape=jax.ShapeDtypeStruct((NSLOT, D_MODEL), jnp.float32),
    )(be, xs, w1, w1, w2)

    return _k3_combine(ys, posa, posb, wa, wb)


# speedup vs baseline: 1.3372x; 1.3372x over previous
import functools

import jax
import jax.numpy as jnp
import numpy as np
from jax import lax
from jax.experimental import pallas as pl
from jax.experimental.pallas import tpu as pltpu
from jax.experimental.pallas import tpu_sc as plsc

E = 8
D_MODEL = 1024
D_FF = 2048
T = 1024
NC = 2
NSUB = 16
L = 16
NW = NC * NSUB
TPW = T // NW
BM = 128
NB = (2 * T) // BM + E - 1
NSLOT = NB * BM

_MESH = plsc.VectorSubcoreMesh(core_axis_name="c", subcore_axis_name="s")


def _ones01():
    return jnp.full((L,), 1, jnp.int32), jnp.zeros((L,), jnp.int32)


def _wid():
    return lax.axis_index("s") * NC + lax.axis_index("c")


def _splat(v, i):
    idx = jnp.full((L,), i, jnp.int32)
    return v.at[idx].get(mode="promise_in_bounds")


def _cumsum(v):
    iota = lax.iota(jnp.int32, L)
    zero = jnp.zeros((L,), v.dtype)
    acc = v
    for d in (1, 2, 4, 8):
        sh = acc.at[jnp.maximum(iota - d, 0)].get(mode="promise_in_bounds")
        acc = acc + jnp.where(iota >= d, sh, zero)
    return acc


def _cummax(v):
    iota = lax.iota(jnp.int32, L)
    acc = v
    for d in (1, 2, 4, 8):
        sh = acc.at[jnp.maximum(iota - d, 0)].get(mode="promise_in_bounds")
        acc = jnp.where(iota >= d, jnp.maximum(acc, sh), acc)
    return acc


def _routing_groups(gv_ref):
    out = []
    for g in range(2):
        ge = [gv_ref[pl.ds(e * TPW + g * L, L)] for e in range(E)]
        m1 = ge[0]
        i1 = jnp.zeros((L,), jnp.int32)
        for e in range(1, E):
            upd = ge[e] > m1
            m1 = jnp.where(upd, ge[e], m1)
            i1 = jnp.where(upd, e, i1)
        m2 = jnp.full((L,), -jnp.inf, jnp.float32)
        i2 = jnp.zeros((L,), jnp.int32)
        for e in range(E):
            cand = jnp.where(i1 == e, -jnp.inf, ge[e])
            upd = cand > m2
            m2 = jnp.where(upd, cand, m2)
            i2 = jnp.where(upd, e, i2)
        wa = 1.0 / (1.0 + jnp.exp(m2 - m1))
        out.append((i1, i2, wa, 1.0 - wa))
    return out


@functools.partial(
    pl.kernel,
    out_type=jax.ShapeDtypeStruct((NW * L,), jnp.int32),
    mesh=_MESH,
    scratch_types=[
        pltpu.VMEM((E * TPW,), jnp.float32),
        pltpu.VMEM((L,), jnp.int32),
    ],
)
def _k1a_counts(gt_hbm, counts_hbm, gv, cv):
    w = _wid()
    for e in range(E):
        pltpu.sync_copy(gt_hbm.at[pl.ds(e * T + w * TPW, TPW)],
                        gv.at[pl.ds(e * TPW, TPW)])
    groups = _routing_groups(gv)
    iota = lax.iota(jnp.int32, L)
    counts = jnp.zeros((L,), jnp.int32)
    one, zero = _ones01()
    for (i1, i2, _, _) in groups:
        for v in (i1, i2):
            for a in range(L):
                counts = counts + jnp.where(_splat(v, a) == iota, one, zero)
    cv[...] = counts
    pltpu.sync_copy(cv, counts_hbm.at[pl.ds(w * L, L)])


@functools.partial(
    pl.kernel,
    out_type=[
        jax.ShapeDtypeStruct((NSLOT, D_MODEL), jnp.float32),
        jax.ShapeDtypeStruct((L,), jnp.int32),
        jax.ShapeDtypeStruct((L,), jnp.int32),
        jax.ShapeDtypeStruct((T,), jnp.int32),
        jax.ShapeDtypeStruct((T,), jnp.int32),
        jax.ShapeDtypeStruct((T,), jnp.float32),
        jax.ShapeDtypeStruct((T,), jnp.float32),
    ],
    mesh=_MESH,
    scratch_types=[
        pltpu.VMEM((E * TPW,), jnp.float32),
        pltpu.VMEM((NW * L,), jnp.int32),
        pltpu.VMEM((TPW, D_MODEL), jnp.float32),
        pltpu.VMEM((TPW,), jnp.int32),
        pltpu.VMEM((TPW,), jnp.int32),
        pltpu.VMEM((TPW,), jnp.float32),
        pltpu.VMEM((TPW,), jnp.float32),
        pltpu.VMEM((L,), jnp.int32),
        pltpu.VMEM((L,), jnp.int32),
        pltpu.SemaphoreType.DMA,
    ],
)
def _k1b_dispatch(gt_hbm, counts_hbm, x_hbm,
                  xs_hbm, sb_hbm, nb_hbm, posa_hbm, posb_hbm, wa_hbm, wb_hbm,
                  gv, cnts, xrows, pA, pB, wav, wbv, sbv, nbv, sem):
    w = _wid()
    iota = lax.iota(jnp.int32, L)
    for e in range(E):
        pltpu.sync_copy(gt_hbm.at[pl.ds(e * T + w * TPW, TPW)],
                        gv.at[pl.ds(e * TPW, TPW)])
    pltpu.sync_copy(counts_hbm, cnts)
    groups = _routing_groups(gv)

    def _pref(i, acc):
        return acc + cnts[pl.ds(i * L, L)]

    excl = lax.fori_loop(0, w, _pref, jnp.zeros((L,), jnp.int32))
    c_tot = excl + lax.fori_loop(w, NW, _pref, jnp.zeros((L,), jnp.int32))

    nb = (c_tot + (BM - 1)) >> 7
    startblk = _cumsum(nb) - nb
    off = startblk * BM + excl

    for g in range(2):
        i1, i2, wa, wb = groups[g]
        wav[pl.ds(g * L, L)] = wa
        wbv[pl.ds(g * L, L)] = wb
        posg = []
        for idx in (i1, i2):
            pos = jnp.zeros((L,), jnp.int32)
            one, zero = _ones01()
            for e in range(E):
                m = idx == e
                r = _cumsum(jnp.where(m, one, zero))
                pos = jnp.where(m, _splat(off, e) + r - 1, pos)
                off = jnp.where(iota == e, off + _splat(r, L - 1), off)
            posg.append(pos)
        pA[pl.ds(g * L, L)] = posg[0]
        pB[pl.ds(g * L, L)] = posg[1]

    base = w * TPW
    pltpu.sync_copy(wav, wa_hbm.at[pl.ds(base, TPW)])
    pltpu.sync_copy(wbv, wb_hbm.at[pl.ds(base, TPW)])
    pltpu.sync_copy(pA, posa_hbm.at[pl.ds(base, TPW)])
    pltpu.sync_copy(pB, posb_hbm.at[pl.ds(base, TPW)])

    pltpu.sync_copy(x_hbm.at[pl.ds(base, TPW)], xrows)
    pltpu.async_copy(xrows, xs_hbm.at[pA], sem).wait()
    pltpu.async_copy(xrows, xs_hbm.at[pB], sem).wait()

    sbv[...] = startblk
    nbv[...] = nb

    @pl.when(w == 0)
    def _():
        pltpu.sync_copy(sbv, sb_hbm)
        pltpu.sync_copy(nbv, nb_hbm)


def _k2_body(sb_ref, nb_ref, xs_ref, w1g_ref, w1u_ref, w2_ref, ys_ref):
    e = pl.program_id(0)
    f = pl.program_id(1)
    sb = sb_ref[e]

    def step(b, _):
        rows = pl.ds((sb + b) * BM, BM)
        xb = xs_ref[rows, :]
        g = jnp.dot(xb, w1g_ref[0], preferred_element_type=jnp.float32)
        u = jnp.dot(xb, w1u_ref[0], preferred_element_type=jnp.float32)
        a = g * jax.lax.logistic(g) * u
        y = jnp.dot(a, w2_ref[0], preferred_element_type=jnp.float32)

        @pl.when(f == 0)
        def _():
            ys_ref[rows, :] = y

        @pl.when(f != 0)
        def _():
            ys_ref[rows, :] += y

        return 0

    lax.fori_loop(0, nb_ref[e], step, 0)


@functools.partial(
    pl.kernel,
    out_type=jax.ShapeDtypeStruct((T, D_MODEL), jnp.float32),
    mesh=_MESH,
    scratch_types=[
        pltpu.VMEM((TPW,), jnp.int32),
        pltpu.VMEM((TPW,), jnp.int32),
        pltpu.VMEM((TPW,), jnp.float32),
        pltpu.VMEM((TPW,), jnp.float32),
        pltpu.VMEM((TPW, D_MODEL), jnp.float32),
        pltpu.VMEM((TPW, D_MODEL), jnp.float32),
        pltpu.VMEM((TPW, D_MODEL), jnp.float32),
        pltpu.SemaphoreType.DMA,
    ],
)
def _k3_combine(ys_hbm, posa_hbm, posb_hbm, wa_hbm, wb_hbm, out_hbm,
                pA, pB, wav, wbv, rowsA, rowsB, orows, sem):
    w = _wid()
    base = w * TPW
    pltpu.sync_copy(posa_hbm.at[pl.ds(base, TPW)], pA)
    pltpu.sync_copy(posb_hbm.at[pl.ds(base, TPW)], pB)
    pltpu.sync_copy(wa_hbm.at[pl.ds(base, TPW)], wav)
    pltpu.sync_copy(wb_hbm.at[pl.ds(base, TPW)], wbv)
    pltpu.async_copy(ys_hbm.at[pA], rowsA, sem).wait()
    pltpu.async_copy(ys_hbm.at[pB], rowsB, sem).wait()
    for t in range(TPW):
        wac = wav[pl.ds((t // L) * L, L)]
        wbc = wbv[pl.ds((t // L) * L, L)]
        sa = _splat(wac, t % L)
        sb = _splat(wbc, t % L)

        def _chunks(c, _, t=t, sa=sa, sb=sb):
            for k in range(8):
                s = pl.ds(c * (8 * L) + k * L, L)
                orows[t, s] = sa * rowsA[t, s] + sb * rowsB[t, s]
            return 0

        lax.fori_loop(0, D_MODEL // (8 * L), _chunks, 0)
    pltpu.sync_copy(orows, out_hbm.at[pl.ds(base, TPW)])


def kernel(x, gating_output, w1, w2):
    gt = gating_output.T.reshape(-1)
    counts = _k1a_counts(gt)
    xs, sb, nb, posa, posb, wa, wb = _k1b_dispatch(gt, counts, x)

    NF = 2
    BF = D_FF // NF
    grid_spec = pltpu.PrefetchScalarGridSpec(
        num_scalar_prefetch=2,
        grid=(E, NF),
        in_specs=[
            pl.BlockSpec((NSLOT, D_MODEL), lambda e, f, sb_r, nb_r: (0, 0)),
            pl.BlockSpec((1, D_MODEL, BF), lambda e, f, sb_r, nb_r: (e, 0, f)),
            pl.BlockSpec((1, D_MODEL, BF), lambda e, f, sb_r, nb_r: (e, 0, NF + f)),
            pl.BlockSpec((1, BF, D_MODEL), lambda e, f, sb_r, nb_r: (e, f, 0)),
        ],
        out_specs=pl.BlockSpec((NSLOT, D_MODEL), lambda e, f, sb_r, nb_r: (0, 0)),
    )
    ys = pl.pallas_call(
        _k2_body,
        grid_spec=grid_spec,
        out_shape=jax.ShapeDtypeStruct((NSLOT, D_MODEL), jnp.float32),
    )(sb, nb, xs, w1, w1, w2)

    return _k3_combine(ys, posa, posb, wa, wb)
